# baseline (speedup 1.0000x reference)
import jax
import jax.numpy as jnp
from jax import lax
from jax.experimental import pallas as pl
from jax.experimental.pallas import tpu as pltpu
from jax.experimental.pallas import tpu_sc as plsc

NUM_CORES = 2
NUM_SUBCORES = 16
LANES = 16
NUM_WORKERS = NUM_CORES * NUM_SUBCORES

BATCH = 16384
EMB_DIM = 32
NUM_ROWS = 1000000
B_PER_W = BATCH // NUM_WORKERS
IDX_CHUNK = 128
N_CHUNKS = B_PER_W // IDX_CHUNK
N_GROUPS = B_PER_W // LANES


def _bpr_kernel(users_hbm, item_i_hbm, item_j_hbm, ut_hbm, it_hbm,
                w_hbm, b_hbm, out_pos_hbm, out_neg_hbm,
                uidx_v, iidx_v, jidx_v, u_vals, i_vals, j_vals,
                w_v, b_v, outp_v, outn_v, sem):
    wid = lax.axis_index("s") * NUM_CORES + lax.axis_index("c")
    base = wid * B_PER_W

    pltpu.sync_copy(users_hbm.at[pl.ds(base, B_PER_W)], uidx_v)
    pltpu.sync_copy(item_i_hbm.at[pl.ds(base, B_PER_W)], iidx_v)
    pltpu.sync_copy(item_j_hbm.at[pl.ds(base, B_PER_W)], jidx_v)
    pltpu.sync_copy(w_hbm, w_v)
    pltpu.sync_copy(b_hbm, b_v)

    copies = []
    for d in range(EMB_DIM):
        for c in range(N_CHUNKS):
            sl = pl.ds(c * IDX_CHUNK, IDX_CHUNK)
            copies.append(pltpu.async_copy(
                ut_hbm.at[d].at[uidx_v.at[sl]], u_vals.at[d, sl], sem))
            copies.append(pltpu.async_copy(
                it_hbm.at[d].at[iidx_v.at[sl]], i_vals.at[d, sl], sem))
            copies.append(pltpu.async_copy(
                it_hbm.at[d].at[jidx_v.at[sl]], j_vals.at[d, sl], sem))
    for cp in copies:
        cp.wait()

    bias = b_v[pl.ds(0, LANES)]
    w_lo = w_v[pl.ds(0, LANES)]
    w_hi = w_v[pl.ds(LANES, LANES)]

    def group_body(g, carry):
        sl = pl.ds(g * LANES, LANES)
        acc_p = bias
        acc_n = bias
        for d in range(EMB_DIM):
            wd = (w_lo if d < LANES else w_hi)[d % LANES]
            uw = u_vals[d, sl] * wd
            acc_p = acc_p + uw * i_vals[d, sl]
            acc_n = acc_n + uw * j_vals[d, sl]
        outp_v[sl] = acc_p
        outn_v[sl] = acc_n
        return carry

    lax.fori_loop(0, N_GROUPS, group_body, 0)

    pltpu.sync_copy(outp_v, out_pos_hbm.at[pl.ds(base, B_PER_W)])
    pltpu.sync_copy(outn_v, out_neg_hbm.at[pl.ds(base, B_PER_W)])


@jax.jit
def kernel(users, item_i, item_j, user_emb, item_emb, W, b):
    mesh = plsc.VectorSubcoreMesh(core_axis_name="c", subcore_axis_name="s")
    w_flat = W.reshape(EMB_DIM).astype(jnp.float32)
    b_vec = jnp.broadcast_to(b.reshape(1), (LANES,)).astype(jnp.float32)

    run = pl.kernel(
        _bpr_kernel,
        out_type=(
            jax.ShapeDtypeStruct((BATCH,), jnp.float32),
            jax.ShapeDtypeStruct((BATCH,), jnp.float32),
        ),
        mesh=mesh,
        compiler_params=pltpu.CompilerParams(use_tc_tiling_on_sc=False),
        scratch_types=[
            pltpu.VMEM((B_PER_W,), jnp.int32),
            pltpu.VMEM((B_PER_W,), jnp.int32),
            pltpu.VMEM((B_PER_W,), jnp.int32),
            pltpu.VMEM((EMB_DIM, B_PER_W), jnp.float32),
            pltpu.VMEM((EMB_DIM, B_PER_W), jnp.float32),
            pltpu.VMEM((EMB_DIM, B_PER_W), jnp.float32),
            pltpu.VMEM((EMB_DIM,), jnp.float32),
            pltpu.VMEM((LANES,), jnp.float32),
            pltpu.VMEM((B_PER_W,), jnp.float32),
            pltpu.VMEM((B_PER_W,), jnp.float32),
            pltpu.SemaphoreType.DMA,
        ],
        name="bpr_sc",
    )
    out_pos, out_neg = run(
        users.astype(jnp.int32), item_i.astype(jnp.int32),
        item_j.astype(jnp.int32), user_emb.T, item_emb.T, w_flat, b_vec)
    return out_pos, out_neg


# speedup vs baseline: 17.4216x; 17.4216x over previous
import jax
import jax.numpy as jnp
from jax import lax
from jax.experimental import pallas as pl
from jax.experimental.pallas import tpu as pltpu
from jax.experimental.pallas import tpu_sc as plsc

NUM_CORES = 2
NUM_SUBCORES = 16
LANES = 16
NUM_WORKERS = NUM_CORES * NUM_SUBCORES

BATCH = 16384
EMB_DIM = 32
NUM_ROWS = 1000000
B_PER_W = BATCH // NUM_WORKERS
N_QUADS = B_PER_W // 4
FETCH_W = 128
FETCH_MASK = FETCH_W - 1


def _bpr_kernel(qidx_hbm, ut_hbm, it_hbm, w_hbm, b_hbm,
                out_pos_hbm, out_neg_hbm,
                qidx_v, u_buf, i_buf, j_buf, w_v, b_v, outp_v, outn_v, sem):
    wid = lax.axis_index("s") * NUM_CORES + lax.axis_index("c")
    base = wid * B_PER_W

    pltpu.sync_copy(qidx_hbm.at[wid], qidx_v)
    pltpu.sync_copy(w_hbm, w_v)
    pltpu.sync_copy(b_hbm, b_v)

    w_lo = w_v[pl.ds(0, LANES)]
    w_hi = w_v[pl.ds(LANES, LANES)]
    b_s = b_v[pl.ds(0, LANES)][0]
    iota16 = lax.iota(jnp.int32, LANES)
    row_lo = iota16
    row_hi = iota16 + LANES

    def fire(q, slot):
        qv = qidx_v[q, pl.ds(0, LANES)]
        for e in range(4):
            for buf, lane, tab in ((u_buf, e, ut_hbm),
                                   (i_buf, 4 + e, it_hbm),
                                   (j_buf, 8 + e, it_hbm)):
                x = qv[lane]
                xa = pl.multiple_of(x & ~FETCH_MASK, 128)
                pltpu.async_copy(
                    tab.at[:, pl.ds(xa, FETCH_W)], buf.at[slot, e], sem)

    def drain(slot):
        for buf in (u_buf, i_buf, j_buf):
            for e in range(4):
                pltpu.make_async_copy(
                    ut_hbm.at[:, pl.ds(0, FETCH_W)], buf.at[slot, e], sem
                ).wait()

    def compute(q, slot, acc_p, acc_n):
        qv = qidx_v[q, pl.ds(0, LANES)]
        for e in range(4):
            xu = qv[e]
            xi = qv[4 + e]
            xj = qv[8 + e]
            cu = jnp.full((LANES,), xu & FETCH_MASK, jnp.int32)
            ci = jnp.full((LANES,), xi & FETCH_MASK, jnp.int32)
            cj = jnp.full((LANES,), xj & FETCH_MASK, jnp.int32)
            u0 = plsc.load_gather(u_buf.at[slot, e], [row_lo, cu])
            u1 = plsc.load_gather(u_buf.at[slot, e], [row_hi, cu])
            i0 = plsc.load_gather(i_buf.at[slot, e], [row_lo, ci])
            i1 = plsc.load_gather(i_buf.at[slot, e], [row_hi, ci])
            j0 = plsc.load_gather(j_buf.at[slot, e], [row_lo, cj])
            j1 = plsc.load_gather(j_buf.at[slot, e], [row_hi, cj])
            uw0 = u0 * w_lo
            uw1 = u1 * w_hi
            pos = jnp.sum(uw0 * i0 + uw1 * i1) + b_s
            neg = jnp.sum(uw0 * j0 + uw1 * j1) + b_s
            onehot = iota16 == ((q % 4) * 4 + e)
            acc_p = jnp.where(onehot, pos, acc_p)
            acc_n = jnp.where(onehot, neg, acc_n)
        return acc_p, acc_n

    fire(0, 0)

    def body(q2, carry):
        acc_p, acc_n = carry
        for par in range(2):
            q = q2 * 2 + par

            @pl.when(q + 1 < N_QUADS)
            def _():
                fire(q + 1, 1 - par)

            drain(par)
            acc_p, acc_n = compute(q, par, acc_p, acc_n)

            @pl.when(q % 4 == 3)
            def _():
                outp_v[pl.ds((q // 4) * LANES, LANES)] = acc_p
                outn_v[pl.ds((q // 4) * LANES, LANES)] = acc_n

        return acc_p, acc_n

    zeros = jnp.zeros((LANES,), jnp.float32)
    lax.fori_loop(0, N_QUADS // 2, body, (zeros, zeros))

    pltpu.sync_copy(outp_v, out_pos_hbm.at[pl.ds(base, B_PER_W)])
    pltpu.sync_copy(outn_v, out_neg_hbm.at[pl.ds(base, B_PER_W)])


@jax.jit
def kernel(users, item_i, item_j, user_emb, item_emb, W, b):
    mesh = plsc.VectorSubcoreMesh(core_axis_name="c", subcore_axis_name="s")
    w_flat = W.reshape(EMB_DIM).astype(jnp.float32)
    b_vec = jnp.broadcast_to(b.reshape(1), (LANES,)).astype(jnp.float32)

    u4 = users.astype(jnp.int32).reshape(NUM_WORKERS, N_QUADS, 4)
    i4 = item_i.astype(jnp.int32).reshape(NUM_WORKERS, N_QUADS, 4)
    j4 = item_j.astype(jnp.int32).reshape(NUM_WORKERS, N_QUADS, 4)
    pad = jnp.zeros_like(u4)
    qidx = jnp.concatenate([u4, i4, j4, pad], axis=-1)

    run = pl.kernel(
        _bpr_kernel,
        out_type=(
            jax.ShapeDtypeStruct((BATCH,), jnp.float32),
            jax.ShapeDtypeStruct((BATCH,), jnp.float32),
        ),
        mesh=mesh,
        compiler_params=pltpu.CompilerParams(needs_layout_passes=False),
        scratch_types=[
            pltpu.VMEM((N_QUADS, LANES), jnp.int32),
            pltpu.VMEM((2, 4, EMB_DIM, FETCH_W), jnp.float32),
            pltpu.VMEM((2, 4, EMB_DIM, FETCH_W), jnp.float32),
            pltpu.VMEM((2, 4, EMB_DIM, FETCH_W), jnp.float32),
            pltpu.VMEM((EMB_DIM,), jnp.float32),
            pltpu.VMEM((LANES,), jnp.float32),
            pltpu.VMEM((B_PER_W,), jnp.float32),
            pltpu.VMEM((B_PER_W,), jnp.float32),
            pltpu.SemaphoreType.DMA,
        ],
        name="bpr_sc",
    )
    out_pos, out_neg = run(qidx, user_emb.T, item_emb.T, w_flat, b_vec)
    return out_pos, out_neg


# speedup vs baseline: 17.4233x; 1.0001x over previous
import jax
import jax.numpy as jnp
from jax import lax
from jax.experimental import pallas as pl
from jax.experimental.pallas import tpu as pltpu
from jax.experimental.pallas import tpu_sc as plsc

NUM_CORES = 2
NUM_SUBCORES = 16
LANES = 16
NUM_WORKERS = NUM_CORES * NUM_SUBCORES

BATCH = 16384
EMB_DIM = 32
NUM_ROWS = 1000000
B_PER_W = BATCH // NUM_WORKERS
N_QUADS = B_PER_W // 4
FETCH_W = 128
FETCH_MASK = FETCH_W - 1


def _bpr_kernel(qidx_hbm, ut_hbm, it_hbm, w_hbm, b_hbm,
                out_pos_hbm, out_neg_hbm,
                qidx_v, u_buf, i_buf, j_buf, w_v, b_v, outp_v, outn_v, sem):
    wid = lax.axis_index("s") * NUM_CORES + lax.axis_index("c")
    base = wid * B_PER_W

    pltpu.sync_copy(qidx_hbm.at[wid], qidx_v)
    pltpu.sync_copy(w_hbm, w_v)
    pltpu.sync_copy(b_hbm, b_v)

    w_lo = w_v[pl.ds(0, LANES)]
    w_hi = w_v[pl.ds(LANES, LANES)]
    b_s = b_v[pl.ds(0, LANES)][0]
    iota16 = lax.iota(jnp.int32, LANES)
    row_lo = iota16
    row_hi = iota16 + LANES

    def fire(q, slot):
        qv = qidx_v[q, pl.ds(0, LANES)]
        for e in range(4):
            for buf, lane, tab in ((u_buf, e, ut_hbm),
                                   (i_buf, 4 + e, it_hbm),
                                   (j_buf, 8 + e, it_hbm)):
                x = qv[lane]
                xa = pl.multiple_of(x & ~FETCH_MASK, 128)
                for t in range(4):
                    pltpu.async_copy(
                        tab.at[pl.ds(t * 8, 8), pl.ds(xa, FETCH_W)],
                        buf.at[slot, e, pl.ds(t * 8, 8)], sem)

    def drain(slot):
        for buf in (u_buf, i_buf, j_buf):
            for e in range(4):
                for t in range(4):
                    pltpu.make_async_copy(
                        ut_hbm.at[pl.ds(0, 8), pl.ds(0, FETCH_W)],
                        buf.at[slot, e, pl.ds(t * 8, 8)], sem
                    ).wait()

    def compute(q, slot, acc_p, acc_n):
        qv = qidx_v[q, pl.ds(0, LANES)]
        for e in range(4):
            xu = qv[e]
            xi = qv[4 + e]
            xj = qv[8 + e]
            cu = jnp.full((LANES,), xu & FETCH_MASK, jnp.int32)
            ci = jnp.full((LANES,), xi & FETCH_MASK, jnp.int32)
            cj = jnp.full((LANES,), xj & FETCH_MASK, jnp.int32)
            u0 = plsc.load_gather(u_buf.at[slot, e], [row_lo, cu])
            u1 = plsc.load_gather(u_buf.at[slot, e], [row_hi, cu])
            i0 = plsc.load_gather(i_buf.at[slot, e], [row_lo, ci])
            i1 = plsc.load_gather(i_buf.at[slot, e], [row_hi, ci])
            j0 = plsc.load_gather(j_buf.at[slot, e], [row_lo, cj])
            j1 = plsc.load_gather(j_buf.at[slot, e], [row_hi, cj])
            uw0 = u0 * w_lo
            uw1 = u1 * w_hi
            pos = jnp.sum(uw0 * i0 + uw1 * i1) + b_s
            neg = jnp.sum(uw0 * j0 + uw1 * j1) + b_s
            onehot = iota16 == ((q % 4) * 4 + e)
            acc_p = jnp.where(onehot, pos, acc_p)
            acc_n = jnp.where(onehot, neg, acc_n)
        return acc_p, acc_n

    fire(0, 0)

    def body(q2, carry):
        acc_p, acc_n = carry
        for par in range(2):
            q = q2 * 2 + par

            @pl.when(q + 1 < N_QUADS)
            def _():
                fire(q + 1, 1 - par)

            drain(par)
            acc_p, acc_n = compute(q, par, acc_p, acc_n)

            @pl.when(q % 4 == 3)
            def _():
                outp_v[pl.ds((q // 4) * LANES, LANES)] = acc_p
                outn_v[pl.ds((q // 4) * LANES, LANES)] = acc_n

        return acc_p, acc_n

    zeros = jnp.zeros((LANES,), jnp.float32)
    lax.fori_loop(0, N_QUADS // 2, body, (zeros, zeros))

    pltpu.sync_copy(outp_v, out_pos_hbm.at[pl.ds(base, B_PER_W)])
    pltpu.sync_copy(outn_v, out_neg_hbm.at[pl.ds(base, B_PER_W)])


@jax.jit
def kernel(users, item_i, item_j, user_emb, item_emb, W, b):
    mesh = plsc.VectorSubcoreMesh(core_axis_name="c", subcore_axis_name="s")
    w_flat = W.reshape(EMB_DIM).astype(jnp.float32)
    b_vec = jnp.broadcast_to(b.reshape(1), (LANES,)).astype(jnp.float32)

    u4 = users.astype(jnp.int32).reshape(NUM_WORKERS, N_QUADS, 4)
    i4 = item_i.astype(jnp.int32).reshape(NUM_WORKERS, N_QUADS, 4)
    j4 = item_j.astype(jnp.int32).reshape(NUM_WORKERS, N_QUADS, 4)
    pad = jnp.zeros_like(u4)
    qidx = jnp.concatenate([u4, i4, j4, pad], axis=-1)

    run = pl.kernel(
        _bpr_kernel,
        out_type=(
            jax.ShapeDtypeStruct((BATCH,), jnp.float32),
            jax.ShapeDtypeStruct((BATCH,), jnp.float32),
        ),
        mesh=mesh,
        compiler_params=pltpu.CompilerParams(needs_layout_passes=False),
        scratch_types=[
            pltpu.VMEM((N_QUADS, LANES), jnp.int32),
            pltpu.VMEM((2, 4, EMB_DIM, FETCH_W), jnp.float32),
            pltpu.VMEM((2, 4, EMB_DIM, FETCH_W), jnp.float32),
            pltpu.VMEM((2, 4, EMB_DIM, FETCH_W), jnp.float32),
            pltpu.VMEM((EMB_DIM,), jnp.float32),
            pltpu.VMEM((LANES,), jnp.float32),
            pltpu.VMEM((B_PER_W,), jnp.float32),
            pltpu.VMEM((B_PER_W,), jnp.float32),
            pltpu.SemaphoreType.DMA,
        ],
        name="bpr_sc",
    )
    out_pos, out_neg = run(qidx, user_emb.T, item_emb.T, w_flat, b_vec)
    return out_pos, out_neg


# speedup vs baseline: 17.4241x; 1.0000x over previous
import jax
import jax.numpy as jnp
from jax import lax
from jax.experimental import pallas as pl
from jax.experimental.pallas import tpu as pltpu
from jax.experimental.pallas import tpu_sc as plsc

NUM_CORES = 2
NUM_SUBCORES = 16
LANES = 16
NUM_WORKERS = NUM_CORES * NUM_SUBCORES

BATCH = 16384
EMB_DIM = 32
NUM_ROWS = 1000000
B_PER_W = BATCH // NUM_WORKERS
N_QUADS = B_PER_W // 4
FETCH_W = 128
FETCH_MASK = FETCH_W - 1


def _bpr_kernel(qidx_hbm, ut_hbm, it_hbm, w_hbm, b_hbm,
                out_pos_hbm, out_neg_hbm,
                qidx_v, u_buf, i_buf, j_buf, w_v, b_v, outp_v, outn_v, sem):
    wid = lax.axis_index("s") * NUM_CORES + lax.axis_index("c")
    base = wid * B_PER_W

    pltpu.sync_copy(qidx_hbm.at[wid], qidx_v)
    pltpu.sync_copy(w_hbm, w_v)
    pltpu.sync_copy(b_hbm, b_v)

    w_lo = w_v[pl.ds(0, LANES)]
    w_hi = w_v[pl.ds(LANES, LANES)]
    b_s = b_v[pl.ds(0, LANES)][0]
    iota16 = lax.iota(jnp.int32, LANES)
    row_lo = iota16
    row_hi = iota16 + LANES

    def fire(q, slot):
        qv = qidx_v[q, pl.ds(0, LANES)]
        for e in range(4):
            for buf, lane, tab in ((u_buf, e, ut_hbm),
                                   (i_buf, 4 + e, it_hbm),
                                   (j_buf, 8 + e, it_hbm)):
                x = qv[lane]
                xa = pl.multiple_of(x & ~FETCH_MASK, 128)
                pltpu.async_copy(
                    tab.at[:, pl.ds(xa, FETCH_W)], buf.at[slot, e], sem)

    def drain(slot):
        for buf in (u_buf, i_buf, j_buf):
            for e in range(4):
                pltpu.make_async_copy(
                    ut_hbm.at[:, pl.ds(0, FETCH_W)], buf.at[slot, e], sem
                ).wait()

    def compute(q, slot, acc_p, acc_n):
        qv = qidx_v[q, pl.ds(0, LANES)]
        for e in range(4):
            xu = qv[e]
            xi = qv[4 + e]
            xj = qv[8 + e]
            cu = jnp.full((LANES,), xu & FETCH_MASK, jnp.int32)
            ci = jnp.full((LANES,), xi & FETCH_MASK, jnp.int32)
            cj = jnp.full((LANES,), xj & FETCH_MASK, jnp.int32)
            u0 = plsc.load_gather(u_buf.at[slot, e], [row_lo, cu])
            u1 = plsc.load_gather(u_buf.at[slot, e], [row_hi, cu])
            i0 = plsc.load_gather(i_buf.at[slot, e], [row_lo, ci])
            i1 = plsc.load_gather(i_buf.at[slot, e], [row_hi, ci])
            j0 = plsc.load_gather(j_buf.at[slot, e], [row_lo, cj])
            j1 = plsc.load_gather(j_buf.at[slot, e], [row_hi, cj])
            uw0 = u0 * w_lo
            uw1 = u1 * w_hi
            pos = jnp.sum(uw0 * i0 + uw1 * i1) + b_s
            neg = jnp.sum(uw0 * j0 + uw1 * j1) + b_s
            onehot = iota16 == ((q % 4) * 4 + e)
            acc_p = jnp.where(onehot, pos, acc_p)
            acc_n = jnp.where(onehot, neg, acc_n)
        return acc_p, acc_n

    fire(0, 0)

    def body(q2, carry):
        acc_p, acc_n = carry
        for par in range(2):
            q = q2 * 2 + par

            @pl.when(q + 1 < N_QUADS)
            def _():
                fire(q + 1, 1 - par)

            drain(par)
            acc_p, acc_n = compute(q, par, acc_p, acc_n)

            @pl.when(q % 4 == 3)
            def _():
                outp_v[pl.ds((q // 4) * LANES, LANES)] = acc_p
                outn_v[pl.ds((q // 4) * LANES, LANES)] = acc_n

        return acc_p, acc_n

    zeros = jnp.zeros((LANES,), jnp.float32)
    lax.fori_loop(0, N_QUADS // 2, body, (zeros, zeros))

    pltpu.sync_copy(outp_v, out_pos_hbm.at[pl.ds(base, B_PER_W)])
    pltpu.sync_copy(outn_v, out_neg_hbm.at[pl.ds(base, B_PER_W)])


@jax.jit
def kernel(users, item_i, item_j, user_emb, item_emb, W, b):
    mesh = plsc.VectorSubcoreMesh(core_axis_name="c", subcore_axis_name="s")
    w_flat = W.reshape(EMB_DIM).astype(jnp.float32)
    b_vec = jnp.broadcast_to(b.reshape(1), (LANES,)).astype(jnp.float32)

    u4 = users.astype(jnp.int32).reshape(NUM_WORKERS, N_QUADS, 4)
    i4 = item_i.astype(jnp.int32).reshape(NUM_WORKERS, N_QUADS, 4)
    j4 = item_j.astype(jnp.int32).reshape(NUM_WORKERS, N_QUADS, 4)
    pad = jnp.zeros_like(u4)
    qidx = jnp.concatenate([u4, i4, j4, pad], axis=-1)

    run = pl.kernel(
        _bpr_kernel,
        out_type=(
            jax.ShapeDtypeStruct((BATCH,), jnp.float32),
            jax.ShapeDtypeStruct((BATCH,), jnp.float32),
        ),
        mesh=mesh,
        compiler_params=pltpu.CompilerParams(needs_layout_passes=False),
        scratch_types=[
            pltpu.VMEM((N_QUADS, LANES), jnp.int32),
            pltpu.VMEM((2, 4, EMB_DIM, FETCH_W), jnp.float32),
            pltpu.VMEM((2, 4, EMB_DIM, FETCH_W), jnp.float32),
            pltpu.VMEM((2, 4, EMB_DIM, FETCH_W), jnp.float32),
            pltpu.VMEM((EMB_DIM,), jnp.float32),
            pltpu.VMEM((LANES,), jnp.float32),
            pltpu.VMEM((B_PER_W,), jnp.float32),
            pltpu.VMEM((B_PER_W,), jnp.float32),
            pltpu.SemaphoreType.DMA,
        ],
        name="bpr_sc",
    )
    out_pos, out_neg = run(qidx, user_emb.T, item_emb.T, w_flat, b_vec)
    return out_pos, out_neg
